# Initial kernel scaffold; baseline (speedup 1.0000x reference)
import jax
import jax.numpy as jnp
from jax import lax
from jax.experimental import pallas as pl
from jax.experimental.pallas import tpu as pltpu
from jax.experimental.pallas import tpu_sc as plsc

N = 10000
NP = 10016
E0 = 100000
BATCH = 128
NBATCH = 49
EPT = BATCH * NBATCH
EP = 16 * EPT
EPT_B = EP // 32
F = 448
FC = 112
STRIPE = NP // 16
NB = 25
NBS = 400

_MESH = plsc.VectorSubcoreMesh(core_axis_name="c", subcore_axis_name="s")


def _deg_body(dst_hbm, zeros1_hbm, deg_out, dstv, accv):
    c = lax.axis_index("c")
    s = lax.axis_index("s")
    w = c * 16 + s
    pltpu.sync_copy(zeros1_hbm, accv)
    pltpu.sync_copy(dst_hbm.at[pl.ds(w * EPT_B, EPT_B)], dstv)

    def body(k, carry):
        idx = dstv[pl.ds(k * 16, 16)]
        plsc.addupdate_scatter(accv, [idx], jnp.full((16,), 1.0, jnp.float32))
        return carry

    lax.fori_loop(0, EPT_B // 16, body, 0)
    pltpu.sync_copy(accv, deg_out.at[w])


_deg_kernel = pl.kernel(
    _deg_body,
    out_type=jax.ShapeDtypeStruct((32, NP), jnp.float32),
    mesh=_MESH,
    scratch_types=[
        pltpu.VMEM((EPT_B,), jnp.int32),
        pltpu.VMEM((NP,), jnp.float32),
    ],
)


def _mp_body(src_hbm, dst_hbm, mp0, mp1, mp2, mp3, dinv_hbm, zrows_hbm,
             zeros1_hbm, h0_out, h1_out, h2_out, h3_out, cacc_out,
             srcv, dstv, rowsv, dinvv, caccv, gsem, acc_sh):
    c = lax.axis_index("c")
    s = lax.axis_index("s")
    pltpu.sync_copy(src_hbm.at[pl.ds(s * NBATCH, NBATCH)], srcv)
    pltpu.sync_copy(dst_hbm.at[pl.ds(s * NBATCH, NBATCH)], dstv)

    def run_chunk(mp_hbm, h_out):
        pltpu.sync_copy(zrows_hbm, acc_sh.at[pl.ds(s * STRIPE, STRIPE)])
        plsc.subcore_barrier()

        def body(j, carry):
            pltpu.async_copy(mp_hbm.at[srcv.at[j]], rowsv, gsem).wait()
            pltpu.sync_copy(rowsv, acc_sh.at[dstv.at[j]], add=True)
            return carry

        lax.fori_loop(0, NBATCH, body, 0)
        plsc.subcore_barrier()
        pltpu.sync_copy(acc_sh.at[pl.ds(s * STRIPE, STRIPE)],
                        h_out.at[pl.ds(s * STRIPE, STRIPE)])
        plsc.subcore_barrier()

    @pl.when(c == 0)
    def _():
        run_chunk(mp0, h0_out)
        run_chunk(mp1, h1_out)

    @pl.when(c == 1)
    def _():
        run_chunk(mp2, h2_out)
        run_chunk(mp3, h3_out)

    @pl.when(c == 0)
    def _():
        pltpu.sync_copy(zeros1_hbm, caccv)
        pltpu.sync_copy(dinv_hbm, dinvv)

        def body(k, carry):
            row = k // (BATCH // 16)
            col = (k % (BATCH // 16)) * 16
            di = plsc.load_gather(dinvv, [dstv[row, pl.ds(col, 16)]])
            plsc.addupdate_scatter(caccv, [srcv[row, pl.ds(col, 16)]], di)
            return carry

        lax.fori_loop(0, EPT // 16, body, 0)
        pltpu.sync_copy(caccv, cacc_out.at[s])


_mp_kernel = pl.kernel(
    _mp_body,
    out_type=(
        jax.ShapeDtypeStruct((NP, FC), jnp.float32),
        jax.ShapeDtypeStruct((NP, FC), jnp.float32),
        jax.ShapeDtypeStruct((NP, FC), jnp.float32),
        jax.ShapeDtypeStruct((NP, FC), jnp.float32),
        jax.ShapeDtypeStruct((16, NP), jnp.float32),
    ),
    mesh=_MESH,
    scratch_types=[
        pltpu.VMEM((NBATCH, BATCH), jnp.int32),
        pltpu.VMEM((NBATCH, BATCH), jnp.int32),
        pltpu.VMEM((BATCH, FC), jnp.float32),
        pltpu.VMEM((NP,), jnp.float32),
        pltpu.VMEM((NP,), jnp.float32),
        pltpu.SemaphoreType.DMA,
        pltpu.VMEM_SHARED((NP, FC), jnp.float32),
    ],
)


def _dinv_body(dp_ref, dv_ref):
    deg = jnp.sum(dp_ref[...], axis=0, keepdims=True) + 1.0
    dv_ref[...] = lax.rsqrt(deg)


def _mm_body(xr_ref, w1_ref, m_ref):
    m_ref[...] = jnp.dot(xr_ref[...], w1_ref[...],
                         preferred_element_type=jnp.float32)


def _scale_body(m_ref, dinv_ref, mp_ref):
    mp_ref[...] = dinv_ref[0][:, None] * m_ref[...]


def _red_body(h1_ref, mp_ref, dinv_ref, cacc_ref, b1_ref, out_ref):
    i = pl.program_id(0)
    dinv = dinv_ref[0]
    h1 = dinv[:, None] * (h1_ref[...] + mp_ref[...]) + b1_ref[...]
    rl = jnp.maximum(h1, 0.0)
    w = dinv * jnp.sum(cacc_ref[...], axis=0) + dinv * dinv

    @pl.when(i == 0)
    def _():
        out_ref[...] = jnp.zeros_like(out_ref)

    out_ref[...] += jnp.dot(w[None, :], rl, preferred_element_type=jnp.float32)


def _fin_body(r_ref, w2_ref, b2_ref, wfc_ref, bfc_ref, o_ref):
    h = jnp.dot(r_ref[...], w2_ref[...],
                preferred_element_type=jnp.float32) / N + b2_ref[...]
    val = jnp.sum(h * wfc_ref[...]) + bfc_ref[0, 0]
    o_ref[...] = jax.nn.sigmoid(val.reshape(1, 1) / 28.0)


def kernel(x, edge_index, W1, b1, W2, b2, Wfc, bfc):
    src = edge_index[0].astype(jnp.int32)
    dst = edge_index[1].astype(jnp.int32)
    pad = jnp.full((EP - E0,), N, jnp.int32)
    srcp = jnp.concatenate([src, pad])
    dstp = jnp.concatenate([dst, pad])
    src2d = srcp.reshape(16 * NBATCH, BATCH)
    dst2d = dstp.reshape(16 * NBATCH, BATCH)
    zeros1 = jnp.zeros((NP,), jnp.float32)
    zrows = jnp.zeros((STRIPE, FC), jnp.float32)

    degparts = _deg_kernel(dstp, zeros1)

    dinv = pl.pallas_call(
        _dinv_body,
        in_specs=[pl.BlockSpec((32, NP), lambda: (0, 0))],
        out_specs=pl.BlockSpec((1, NP), lambda: (0, 0)),
        out_shape=jax.ShapeDtypeStruct((1, NP), jnp.float32),
    )(degparts)
    dinv1d = dinv.reshape(NP)
    dinv2d = dinv1d[:N].reshape(NB, NBS)

    xr = x.reshape(N * 28, 28)
    m2 = pl.pallas_call(
        _mm_body,
        grid=(NB,),
        in_specs=[pl.BlockSpec((NBS * 28, 28), lambda i: (i, 0)),
                  pl.BlockSpec((28, 16), lambda i: (0, 0))],
        out_specs=pl.BlockSpec((NBS * 28, 16), lambda i: (i, 0)),
        out_shape=jax.ShapeDtypeStruct((N * 28, 16), jnp.float32),
    )(xr, W1)
    m = m2.reshape(N, F)

    mps = []
    for cch in range(4):
        mp_c = pl.pallas_call(
            _scale_body,
            grid=(NB,),
            in_specs=[pl.BlockSpec((NBS, FC), lambda i, cc=cch: (i, cc)),
                      pl.BlockSpec((1, NBS), lambda i: (i, 0))],
            out_specs=pl.BlockSpec((NBS, FC), lambda i: (i, 0)),
            out_shape=jax.ShapeDtypeStruct((NP, FC), jnp.float32),
        )(m, dinv2d)
        mps.append(mp_c)

    h0, h1, h2, h3, caccparts = _mp_kernel(
        src2d, dst2d, mps[0], mps[1], mps[2], mps[3], dinv1d, zrows, zeros1)

    b1tile = jnp.tile(b1.astype(jnp.float32), 28)
    rs = []
    for cch, h_c in enumerate((h0, h1, h2, h3)):
        r_c = pl.pallas_call(
            _red_body,
            grid=(NB,),
            in_specs=[pl.BlockSpec((NBS, FC), lambda i: (i, 0)),
                      pl.BlockSpec((NBS, FC), lambda i: (i, 0)),
                      pl.BlockSpec((1, NBS), lambda i: (i, 0)),
                      pl.BlockSpec((16, NBS), lambda i: (0, i)),
                      pl.BlockSpec((1, FC), lambda i: (0, 0))],
            out_specs=pl.BlockSpec((1, FC), lambda i: (0, 0)),
            out_shape=jax.ShapeDtypeStruct((1, FC), jnp.float32),
        )(h_c, mps[cch], dinv2d, caccparts,
          b1tile[cch * FC:(cch + 1) * FC].reshape(1, FC))
        rs.append(r_c)

    r28 = jnp.concatenate(rs, axis=1).reshape(28, 16)
    out = pl.pallas_call(
        _fin_body,
        in_specs=[pl.BlockSpec((28, 16), lambda: (0, 0)),
                  pl.BlockSpec((16, 32), lambda: (0, 0)),
                  pl.BlockSpec((1, 32), lambda: (0, 0)),
                  pl.BlockSpec((28, 32), lambda: (0, 0)),
                  pl.BlockSpec((1, 1), lambda: (0, 0))],
        out_specs=pl.BlockSpec((1, 1), lambda: (0, 0)),
        out_shape=jax.ShapeDtypeStruct((1, 1), jnp.float32),
    )(r28, W2.astype(jnp.float32), b2.reshape(1, 32),
      Wfc.reshape(28, 32), bfc.reshape(1, 1))
    return out


# speedup vs baseline: 52.8280x; 52.8280x over previous
import jax
import jax.numpy as jnp
from jax import lax
from jax.experimental import pallas as pl
from jax.experimental.pallas import tpu as pltpu
from jax.experimental.pallas import tpu_sc as plsc

N = 10000
NP = 10240
E0 = 100000
BATCH = 128
NBATCH = 49
EPT = BATCH * NBATCH
EP = 16 * EPT
EPT_B = EP // 32
F = 448
FC = 112
STRIPE = NP // 16
NB = 20
NBS = 512

def _deg_body(dst_hbm, zeros1_hbm, deg_out, dstv, accv):
    c = lax.axis_index("c")
    s = lax.axis_index("s")
    w = c * 16 + s
    pltpu.sync_copy(zeros1_hbm, accv)
    pltpu.sync_copy(dst_hbm.at[pl.ds(w * EPT_B, EPT_B)], dstv)

    def body(k, carry):
        idx = dstv[pl.ds(k * 16, 16)]
        plsc.addupdate_scatter(accv, [idx], jnp.full((16,), 1.0, jnp.float32))
        return carry

    lax.fori_loop(0, EPT_B // 16, body, 0)
    pltpu.sync_copy(accv, deg_out.at[w])


def _make_deg_kernel(mesh):
    return pl.kernel(
        _deg_body,
        out_type=jax.ShapeDtypeStruct((32, NP), jnp.float32),
        mesh=mesh,
        scratch_types=[
            pltpu.VMEM((EPT_B,), jnp.int32),
            pltpu.VMEM((NP,), jnp.float32),
        ],
        compiler_params=pltpu.CompilerParams(needs_layout_passes=False),
    )


def _mp_body(src_hbm, dst_hbm, mp0, mp1, mp2, mp3, dinv_hbm, zrows_hbm,
             zeros1_hbm, h0_out, h1_out, h2_out, h3_out, cacc_out,
             srcv, dstv, rowsv, dinvv, caccv, gsem, acc_sh):
    c = lax.axis_index("c")
    s = lax.axis_index("s")
    pltpu.sync_copy(src_hbm.at[s], srcv)
    pltpu.sync_copy(dst_hbm.at[s], dstv)

    def run_chunk(mp_hbm, h_out):
        pltpu.sync_copy(zrows_hbm, acc_sh.at[pl.ds(s * STRIPE, STRIPE)])
        plsc.subcore_barrier()

        def body(j, carry):
            pltpu.async_copy(mp_hbm.at[srcv.at[j]], rowsv, gsem).wait()
            pltpu.sync_copy(rowsv, acc_sh.at[dstv.at[j]], add=True)
            return carry

        lax.fori_loop(0, NBATCH, body, 0)
        plsc.subcore_barrier()
        pltpu.sync_copy(acc_sh.at[pl.ds(s * STRIPE, STRIPE)],
                        h_out.at[pl.ds(s * STRIPE, STRIPE)])
        plsc.subcore_barrier()

    @pl.when(c == 0)
    def _():
        run_chunk(mp0, h0_out)
        run_chunk(mp1, h1_out)

    @pl.when(c == 1)
    def _():
        run_chunk(mp2, h2_out)
        run_chunk(mp3, h3_out)

    @pl.when(c == 0)
    def _():
        pltpu.sync_copy(zeros1_hbm, caccv)
        pltpu.sync_copy(dinv_hbm, dinvv)

        def body(k, carry):
            row = k // (BATCH // 16)
            col = (k % (BATCH // 16)) * 16
            di = plsc.load_gather(dinvv, [dstv[row, pl.ds(col, 16)]])
            plsc.addupdate_scatter(caccv, [srcv[row, pl.ds(col, 16)]], di)
            return carry

        lax.fori_loop(0, EPT // 16, body, 0)
        pltpu.sync_copy(caccv, cacc_out.at[s])


def _make_mp_kernel(mesh):
    return pl.kernel(
        _mp_body,
        out_type=(
            jax.ShapeDtypeStruct((NP, FC), jnp.float32),
            jax.ShapeDtypeStruct((NP, FC), jnp.float32),
            jax.ShapeDtypeStruct((NP, FC), jnp.float32),
            jax.ShapeDtypeStruct((NP, FC), jnp.float32),
            jax.ShapeDtypeStruct((16, NP), jnp.float32),
        ),
        mesh=mesh,
        scratch_types=[
            pltpu.VMEM((NBATCH, BATCH), jnp.int32),
            pltpu.VMEM((NBATCH, BATCH), jnp.int32),
            pltpu.VMEM((BATCH, FC), jnp.float32),
            pltpu.VMEM((NP,), jnp.float32),
            pltpu.VMEM((NP,), jnp.float32),
            pltpu.SemaphoreType.DMA,
            pltpu.VMEM_SHARED((NP, FC), jnp.float32),
        ],
        compiler_params=pltpu.CompilerParams(needs_layout_passes=False,
                                             use_tc_tiling_on_sc=False),
    )


def _dinv_body(dp_ref, dv_ref):
    deg = jnp.sum(dp_ref[...], axis=0, keepdims=True) + 1.0
    dv_ref[...] = lax.rsqrt(deg)


def _mmscale_body(x_ref, w1d_ref, dinv_ref, mp0_ref, mp1_ref, mp2_ref,
                  mp3_ref):
    i = pl.program_id(0)
    dv = dinv_ref[0, pl.ds(i * NBS, NBS)]
    mm = dv[:, None] * jnp.dot(x_ref[...], w1d_ref[...],
                               preferred_element_type=jnp.float32)
    mp0_ref[...] = mm[:, 0 * FC:1 * FC]
    mp1_ref[...] = mm[:, 1 * FC:2 * FC]
    mp2_ref[...] = mm[:, 2 * FC:3 * FC]
    mp3_ref[...] = mm[:, 3 * FC:4 * FC]


def _w_body(cacc_ref, dinv_ref, out_ref):
    dv = dinv_ref[...]
    w = dv * jnp.sum(cacc_ref[...], axis=0, keepdims=True) + dv * dv
    lane = lax.broadcasted_iota(jnp.int32, (1, NP), 1)
    w = jnp.where(lane < N, w, 0.0)
    out_ref[...] = jnp.concatenate([dv, w], axis=0)


def _red_body(h1_ref, mp_ref, scal_ref, b1_ref, out_ref):
    i = pl.program_id(0)
    dinv = scal_ref[0, pl.ds(i * NBS, NBS)]
    w = scal_ref[1, pl.ds(i * NBS, NBS)]
    h1 = dinv[:, None] * (h1_ref[...] + mp_ref[...]) + b1_ref[...]
    rl = jnp.maximum(h1, 0.0)

    @pl.when(i == 0)
    def _():
        out_ref[...] = jnp.zeros_like(out_ref)

    out_ref[...] += jnp.dot(w[None, :], rl, preferred_element_type=jnp.float32)


def _fin_body(r_ref, w2_ref, b2_ref, wfc_ref, bfc_ref, o_ref):
    h = jnp.dot(r_ref[...], w2_ref[...],
                preferred_element_type=jnp.float32) / N + b2_ref[...]
    val = jnp.sum(h * wfc_ref[...]) + bfc_ref[0, 0]
    o_ref[...] = jax.nn.sigmoid(val.reshape(1, 1) / 28.0)


def kernel(x, edge_index, W1, b1, W2, b2, Wfc, bfc):
    src = edge_index[0].astype(jnp.int32)
    dst = edge_index[1].astype(jnp.int32)
    pad = jnp.full((EP - E0,), N, jnp.int32)
    srcp = jnp.concatenate([src, pad])
    dstp = jnp.concatenate([dst, pad])
    src3d = srcp.reshape(16, NBATCH, BATCH)
    dst3d = dstp.reshape(16, NBATCH, BATCH)
    zeros1 = jnp.zeros((NP,), jnp.float32)
    zrows = jnp.zeros((STRIPE, FC), jnp.float32)

    mesh = plsc.VectorSubcoreMesh(core_axis_name="c", subcore_axis_name="s",
                                  num_cores=2, num_subcores=16)
    degparts = _make_deg_kernel(mesh)(dstp, zeros1)

    dinv = pl.pallas_call(
        _dinv_body,
        in_specs=[pl.BlockSpec((32, NP), lambda: (0, 0))],
        out_specs=pl.BlockSpec((1, NP), lambda: (0, 0)),
        out_shape=jax.ShapeDtypeStruct((1, NP), jnp.float32),
    )(degparts)
    dinv1d = dinv.reshape(NP)

    w1d = jnp.kron(jnp.eye(28, dtype=jnp.float32), W1.astype(jnp.float32))
    mp_spec = pl.BlockSpec((NBS, FC), lambda i: (i, 0))
    x_pad = jnp.pad(x.astype(jnp.float32), ((0, NP - N), (0, 0)))
    mps = pl.pallas_call(
        _mmscale_body,
        grid=(NB,),
        in_specs=[pl.BlockSpec((NBS, 784), lambda i: (i, 0)),
                  pl.BlockSpec((784, F), lambda i: (0, 0)),
                  pl.BlockSpec((1, NP), lambda i: (0, 0))],
        out_specs=[mp_spec, mp_spec, mp_spec, mp_spec],
        out_shape=[jax.ShapeDtypeStruct((NP, FC), jnp.float32)] * 4,
    )(x_pad, w1d, dinv)

    h0, h1, h2, h3, caccparts = _make_mp_kernel(mesh)(
        src3d, dst3d, mps[0], mps[1], mps[2], mps[3], dinv1d, zrows, zeros1)

    scal = pl.pallas_call(
        _w_body,
        in_specs=[pl.BlockSpec((16, NP), lambda: (0, 0)),
                  pl.BlockSpec((1, NP), lambda: (0, 0))],
        out_specs=pl.BlockSpec((2, NP), lambda: (0, 0)),
        out_shape=jax.ShapeDtypeStruct((2, NP), jnp.float32),
    )(caccparts, dinv)

    b1tile = jnp.tile(b1.astype(jnp.float32), 28)
    rs = []
    for cch, h_c in enumerate((h0, h1, h2, h3)):
        r_c = pl.pallas_call(
            _red_body,
            grid=(NB,),
            in_specs=[pl.BlockSpec((NBS, FC), lambda i: (i, 0)),
                      pl.BlockSpec((NBS, FC), lambda i: (i, 0)),
                      pl.BlockSpec((2, NP), lambda i: (0, 0)),
                      pl.BlockSpec((1, FC), lambda i: (0, 0))],
            out_specs=pl.BlockSpec((1, FC), lambda i: (0, 0)),
            out_shape=jax.ShapeDtypeStruct((1, FC), jnp.float32),
        )(h_c, mps[cch], scal,
          b1tile[cch * FC:(cch + 1) * FC].reshape(1, FC))
        rs.append(r_c)

    r28 = jnp.concatenate(rs, axis=1).reshape(28, 16)
    out = pl.pallas_call(
        _fin_body,
        in_specs=[pl.BlockSpec((28, 16), lambda: (0, 0)),
                  pl.BlockSpec((16, 32), lambda: (0, 0)),
                  pl.BlockSpec((1, 32), lambda: (0, 0)),
                  pl.BlockSpec((28, 32), lambda: (0, 0)),
                  pl.BlockSpec((1, 1), lambda: (0, 0))],
        out_specs=pl.BlockSpec((1, 1), lambda: (0, 0)),
        out_shape=jax.ShapeDtypeStruct((1, 1), jnp.float32),
    )(r28, W2.astype(jnp.float32), b2.reshape(1, 32),
      Wfc.reshape(28, 32), bfc.reshape(1, 1))
    return out
